# baseline (speedup 1.0000x reference)
import functools

import jax
import jax.numpy as jnp
from jax import lax
from jax.experimental import pallas as pl
from jax.experimental.pallas import tpu as pltpu
from jax.experimental.pallas import tpu_sc as plsc

SMOOTHING_VALUE = 0.1
PAD_TOKEN_ID = 0
TRG_VOCAB_SIZE = 100000
CONFIDENCE_VALUE = 1.0 - SMOOTHING_VALUE
SMOOTH = SMOOTHING_VALUE / (TRG_VOCAB_SIZE - 2)

BATCH = 1024
V = TRG_VOCAB_SIZE
NW = 32
ROWS_PER_W = BATCH // NW
ZCHUNK = 20000
L = 16


def _sc_body(idx_hbm, out_hbm, idx_v, smooth_v, zero_v, off_v, val_v, sem, sem2):
    nc = 2
    wid = lax.axis_index("s") * nc + lax.axis_index("c")
    row0 = wid * ROWS_PER_W

    pltpu.sync_copy(idx_hbm.at[pl.ds(row0, ROWS_PER_W)], idx_v)

    smooth16 = jnp.full((L,), SMOOTH, dtype=jnp.float32)
    zero16 = jnp.zeros((L,), dtype=jnp.float32)

    def fill_smooth(i, _):
        for u in range(10):
            smooth_v[pl.ds((i * 10 + u) * L, L)] = smooth16
        return 0

    lax.fori_loop(0, V // (L * 10), fill_smooth, 0)

    def fill_zero(i, _):
        for u in range(10):
            zero_v[pl.ds((i * 10 + u) * L, L)] = zero16
        return 0

    lax.fori_loop(0, ZCHUNK // (L * 10), fill_zero, 0)

    lanes = lax.broadcasted_iota(jnp.int32, (L,), 0)

    for r in range(ROWS_PER_W):
        half = idx_v[pl.ds((r // L) * L, L)]
        sidx = jnp.squeeze(lax.slice(half, (r % L,), (r % L + 1,)))
        row_off = (row0 + r) * V

        def nonpad(row_off=row_off):
            pltpu.make_async_copy(
                smooth_v, out_hbm.at[pl.ds(row_off, V)], sem
            ).start()

        def pad(row_off=row_off):
            for c in range(V // ZCHUNK):
                pltpu.make_async_copy(
                    zero_v, out_hbm.at[pl.ds(row_off + c * ZCHUNK, ZCHUNK)], sem
                ).start()

        lax.cond(sidx == PAD_TOKEN_ID, pad, nonpad)

    def drain(r, _):
        pltpu.make_async_copy(
            smooth_v, out_hbm.at[pl.ds((row0 + r) * V, V)], sem
        ).wait()
        return 0

    lax.fori_loop(0, ROWS_PER_W, drain, 0)

    for h in range(ROWS_PER_W // L):
        idx16 = idx_v[pl.ds(h * L, L)]
        rows16 = row0 + h * L + lanes
        off_v[pl.ds(h * L, L)] = rows16 * V + idx16
        val_v[pl.ds(h * L, L)] = jnp.where(
            idx16 == PAD_TOKEN_ID, jnp.float32(0.0), jnp.float32(CONFIDENCE_VALUE)
        )

    pltpu.async_copy(val_v, out_hbm.at[off_v], sem2).wait()


@jax.jit
def kernel(trg_token_ids_batch):
    idx = trg_token_ids_batch.astype(jnp.int32).reshape((BATCH,))
    mesh = plsc.VectorSubcoreMesh(core_axis_name="c", subcore_axis_name="s")
    run = pl.kernel(
        _sc_body,
        out_type=jax.ShapeDtypeStruct((BATCH * V,), jnp.float32),
        mesh=mesh,
        scratch_types=[
            pltpu.VMEM((ROWS_PER_W,), jnp.int32),
            pltpu.VMEM((V,), jnp.float32),
            pltpu.VMEM((ZCHUNK,), jnp.float32),
            pltpu.VMEM((ROWS_PER_W,), jnp.int32),
            pltpu.VMEM((ROWS_PER_W,), jnp.float32),
            pltpu.SemaphoreType.DMA,
            pltpu.SemaphoreType.DMA,
        ],
    )
    flat = run(idx)
    return flat.reshape((BATCH, V))
